# Initial kernel scaffold; baseline (speedup 1.0000x reference)
import jax
import jax.numpy as jnp
from jax.experimental import pallas as pl


def kernel(x, token_emb, pos_emb):
    raise NotImplementedError("write your pallas kernel here")


# speedup vs baseline: 2.0624x; 2.0624x over previous
import functools

import jax
import jax.numpy as jnp
from jax import lax
from jax.experimental import pallas as pl
from jax.experimental.pallas import tpu as pltpu
from jax.experimental.pallas import tpu_sc as plsc

L = 16


def _make_kernel(B, T, D, V):
    NC, NS = 2, 16
    NW = NC * NS
    assert B % NW == 0
    rows_per_w = B // NW
    KA = 128
    KB = T - KA
    assert 0 < KB <= 128 and T % 8 == 0 and KA % 8 == 0 and D % L == 0

    mesh = plsc.VectorSubcoreMesh(core_axis_name="c", subcore_axis_name="s")

    @functools.partial(
        pl.kernel,
        mesh=mesh,
        out_type=jax.ShapeDtypeStruct((B * T, D), jnp.float32),
        scratch_types=[
            pltpu.VMEM((T, D), jnp.float32),
            pltpu.VMEM((KA,), jnp.int32),
            pltpu.VMEM((KB,), jnp.int32),
            pltpu.VMEM((KA, D), jnp.float32),
            pltpu.VMEM((KB, D), jnp.float32),
            pltpu.SemaphoreType.DMA,
        ],
        compiler_params=pltpu.CompilerParams(use_tc_tiling_on_sc=False),
    )
    def k(x_hbm, tok_hbm, pos_hbm, out_hbm, pos_v, idx_a, idx_b, rows_a, rows_b, sem):
        wid = lax.axis_index("s") * NC + lax.axis_index("c")
        pltpu.sync_copy(pos_hbm, pos_v)
        base = wid * rows_per_w

        def row_body(r, carry):
            flat = (base + r) * T

            pltpu.sync_copy(x_hbm.at[pl.ds(flat, KA)], idx_a)
            pltpu.async_copy(tok_hbm.at[idx_a], rows_a, sem).wait()

            def add_a(i, c):
                for j in range(D // L):
                    sl = pl.ds(j * L, L)
                    rows_a[i, sl] = rows_a[i, sl] + pos_v[i, sl]
                return c

            lax.fori_loop(0, KA, add_a, 0, unroll=2)
            pltpu.sync_copy(rows_a, out_hbm.at[pl.ds(flat, KA)])

            pltpu.sync_copy(x_hbm.at[pl.ds(flat + KA, KB)], idx_b)
            pltpu.async_copy(tok_hbm.at[idx_b], rows_b, sem).wait()

            def add_b(i, c):
                for j in range(D // L):
                    sl = pl.ds(j * L, L)
                    rows_b[i, sl] = rows_b[i, sl] + pos_v[KA + i, sl]
                return c

            lax.fori_loop(0, KB, add_b, 0, unroll=2)
            pltpu.sync_copy(rows_b, out_hbm.at[pl.ds(flat + KA, KB)])
            return carry

        lax.fori_loop(0, rows_per_w, row_body, 0)

    return k


def kernel(x, token_emb, pos_emb):
    B, T = x.shape
    V, D = token_emb.shape
    x_flat = x.reshape(B * T).astype(jnp.int32)
    k = _make_kernel(B, T, D, V)
    out = k(x_flat, token_emb, pos_emb)
    return out.reshape(B, T, D)


# speedup vs baseline: 2.7320x; 1.3247x over previous
import functools

import jax
import jax.numpy as jnp
from jax import lax
from jax.experimental import pallas as pl
from jax.experimental.pallas import tpu as pltpu
from jax.experimental.pallas import tpu_sc as plsc

L = 16


def _make_kernel(B, T, D, V):
    NC, NS = 2, 16
    NW = NC * NS
    assert B % (2 * NW) == 0
    R = B // NW
    KA = 128
    KB = T - KA
    assert 0 < KB <= 128 and T % 8 == 0 and KA % 8 == 0 and D % L == 0
    assert KA % 4 == 0 and KB % 4 == 0

    mesh = plsc.VectorSubcoreMesh(core_axis_name="c", subcore_axis_name="s")

    @functools.partial(
        pl.kernel,
        mesh=mesh,
        out_type=jax.ShapeDtypeStruct((B * T, D), jnp.float32),
        scratch_types=[
            pltpu.VMEM((T, D), jnp.float32),
            pltpu.VMEM((T,), jnp.int32),
            pltpu.VMEM((T,), jnp.int32),
            pltpu.VMEM((KA, D), jnp.float32),
            pltpu.VMEM((KA, D), jnp.float32),
            pltpu.VMEM((KB, D), jnp.float32),
            pltpu.VMEM((KB, D), jnp.float32),
            pltpu.SemaphoreType.DMA,
            pltpu.SemaphoreType.DMA,
            pltpu.SemaphoreType.DMA,
            pltpu.SemaphoreType.DMA,
            pltpu.SemaphoreType.DMA,
            pltpu.SemaphoreType.DMA,
        ],
        compiler_params=pltpu.CompilerParams(use_tc_tiling_on_sc=False),
    )
    def k(x_hbm, tok_hbm, pos_hbm, out_hbm,
          pos_v, idx0, idx1, bufA0, bufA1, bufB0, bufB1,
          si0, si1, sg0, sg1, ss0, ss1):
        idx = (idx0, idx1)
        bufA = (bufA0, bufA1)
        bufB = (bufB0, bufB1)
        si = (si0, si1)
        sg = (sg0, sg1)
        ss = (ss0, ss1)

        wid = lax.axis_index("s") * NC + lax.axis_index("c")
        base = wid * R

        def flat(r):
            return (base + r) * T

        def idx_cp(r, b):
            return pltpu.make_async_copy(x_hbm.at[pl.ds(flat(r), T)], idx[b], si[b])

        def gather_a(b):
            return pltpu.make_async_copy(
                tok_hbm.at[idx[b].at[pl.ds(0, KA)]], bufA[b], sg[b])

        def gather_b(b):
            return pltpu.make_async_copy(
                tok_hbm.at[idx[b].at[pl.ds(KA, KB)]], bufB[b], sg[b])

        def store_a(r, b):
            return pltpu.make_async_copy(
                bufA[b], out_hbm.at[pl.ds(flat(r), KA)], ss[b])

        def store_b(r, b):
            return pltpu.make_async_copy(
                bufB[b], out_hbm.at[pl.ds(flat(r) + KA, KB)], ss[b])

        pltpu.sync_copy(pos_hbm, pos_v)
        idx_cp(0, 0).start()
        idx_cp(1, 1).start()
        idx_cp(0, 0).wait()
        gather_a(0).start()
        gather_b(0).start()

        def body(i, carry):
            for sub in (0, 1):
                r = 2 * i + sub
                o = 1 - sub
                gather_a(sub).wait()
                gather_b(sub).wait()

                @pl.when(r + 2 < R)
                def _():
                    idx_cp(r + 2, sub).start()

                @pl.when(r >= 1)
                def _():
                    store_a(r - 1, o).wait()
                    store_b(r - 1, o).wait()

                @pl.when(r + 1 < R)
                def _():
                    idx_cp(r + 1, o).wait()
                    gather_a(o).start()
                    gather_b(o).start()

                def add_a(i2, c):
                    for j in range(D // L):
                        sl = pl.ds(j * L, L)
                        bufA[sub][i2, sl] = bufA[sub][i2, sl] + pos_v[i2, sl]
                    return c

                lax.fori_loop(0, KA, add_a, 0, unroll=4)

                def add_b(i2, c):
                    for j in range(D // L):
                        sl = pl.ds(j * L, L)
                        bufB[sub][i2, sl] = bufB[sub][i2, sl] + pos_v[KA + i2, sl]
                    return c

                lax.fori_loop(0, KB, add_b, 0, unroll=4)

                store_a(r, sub).start()
                store_b(r, sub).start()
            return carry

        lax.fori_loop(0, R // 2, body, 0)
        store_a(R - 1, 1).wait()
        store_b(R - 1, 1).wait()

    return k


def kernel(x, token_emb, pos_emb):
    B, T = x.shape
    V, D = token_emb.shape
    x_flat = x.reshape(B * T).astype(jnp.int32)
    k = _make_kernel(B, T, D, V)
    out = k(x_flat, token_emb, pos_emb)
    return out.reshape(B, T, D)


# speedup vs baseline: 4.1528x; 1.5201x over previous
import functools

import jax
import jax.numpy as jnp
from jax import lax
from jax.experimental import pallas as pl
from jax.experimental.pallas import tpu as pltpu
from jax.experimental.pallas import tpu_sc as plsc

CK = 128
NB = 4


def _make_gather(B, T, D, V):
    NC, NS = 2, 16
    NW = NC * NS
    BT = B * T
    assert BT % (NW * 2 * CK) == 0 and D * 2 == 128
    M = BT // (NW * CK)
    assert M % NB == 0

    mesh = plsc.VectorSubcoreMesh(core_axis_name="c", subcore_axis_name="s")

    @functools.partial(
        pl.kernel,
        mesh=mesh,
        out_type=jax.ShapeDtypeStruct((BT // 2, 2 * D), jnp.float32),
        scratch_types=[
            pltpu.VMEM((M * CK,), jnp.int32),
            pltpu.VMEM((CK, D), jnp.float32),
            pltpu.VMEM((CK, D), jnp.float32),
            pltpu.VMEM((CK, D), jnp.float32),
            pltpu.VMEM((CK, D), jnp.float32),
            pltpu.SemaphoreType.DMA,
            pltpu.SemaphoreType.DMA,
            pltpu.SemaphoreType.DMA,
            pltpu.SemaphoreType.DMA,
            pltpu.SemaphoreType.DMA,
            pltpu.SemaphoreType.DMA,
            pltpu.SemaphoreType.DMA,
            pltpu.SemaphoreType.DMA,
        ],
        compiler_params=pltpu.CompilerParams(use_tc_tiling_on_sc=False),
    )
    def k(x_hbm, tok_hbm, out_hbm,
          idx_all, b0, b1, b2, b3,
          sg0, sg1, sg2, sg3, ss0, ss1, ss2, ss3):
        buf = (b0, b1, b2, b3)
        sg = (sg0, sg1, sg2, sg3)
        ss = (ss0, ss1, ss2, ss3)

        wid = lax.axis_index("s") * NC + lax.axis_index("c")

        def gather(c, b):
            return pltpu.make_async_copy(
                tok_hbm.at[idx_all.at[pl.ds(c * CK, CK)]], buf[b], sg[b])

        def store(c, s, i, b):
            row0 = CK * ((M // 2) * wid + 2 * i + (s // 2))
            return pltpu.make_async_copy(
                buf[b],
                out_hbm.at[pl.ds(row0, CK), pl.ds(D * (s % 2), D)],
                ss[b])

        pltpu.sync_copy(x_hbm.at[pl.ds(wid * (M * CK), M * CK)], idx_all)
        gather(0, 0).start()
        gather(1, 1).start()

        def body(i, carry):
            for s in range(NB):
                c = NB * i + s
                b = s
                gather(c, b).wait()

                @pl.when(c >= 2)
                def _():
                    store(c - 2, (s - 2) % NB, i - (1 if s < 2 else 0),
                          (s - 2) % NB).wait()

                @pl.when(c + 2 < M)
                def _():
                    gather(c + 2, (s + 2) % NB).start()

                store(c, s, i, b).start()
            return carry

        lax.fori_loop(0, M // NB, body, 0)
        store(M - 2, 2, M // NB - 1, 2).wait()
        store(M - 1, 3, M // NB - 1, 3).wait()

    return k


def _make_finish(B, T, D):
    BB = 32
    RB = BB * T // 2
    NP = RB // CK

    def body(g_ref, pos_ref, o_ref):
        v = g_ref[...]
        left = v[:, :D].reshape(NP, CK, D)
        right = v[:, D:].reshape(NP, CK, D)
        y = jnp.concatenate([left, right], axis=1)
        y = y.reshape(BB, T, D)
        o_ref[...] = y + pos_ref[...][None, :, :]

    return pl.pallas_call(
        body,
        grid=(B // BB,),
        in_specs=[
            pl.BlockSpec((RB, 2 * D), lambda i: (i, 0)),
            pl.BlockSpec((T, D), lambda i: (0, 0)),
        ],
        out_specs=pl.BlockSpec((BB, T, D), lambda i: (i, 0, 0)),
        out_shape=jax.ShapeDtypeStruct((B, T, D), jnp.float32),
    )


def kernel(x, token_emb, pos_emb):
    B, T = x.shape
    V, D = token_emb.shape
    x_flat = x.reshape(B * T).astype(jnp.int32)
    g = _make_gather(B, T, D, V)(x_flat, token_emb)
    return _make_finish(B, T, D)(g, pos_emb)


# speedup vs baseline: 7.2131x; 1.7369x over previous
import functools

import jax
import jax.numpy as jnp
from jax import lax
from jax.experimental import pallas as pl
from jax.experimental.pallas import tpu as pltpu
from jax.experimental.pallas import tpu_sc as plsc

CK = 128
NB = 4


def _make_gather(B, T, D, V):
    NC, NS = 2, 16
    NW = NC * NS
    BT = B * T
    assert BT % (NW * 2 * CK) == 0 and D * 2 == 128
    M = BT // (NW * CK)
    assert M % NB == 0

    mesh = plsc.VectorSubcoreMesh(core_axis_name="c", subcore_axis_name="s")

    @functools.partial(
        pl.kernel,
        mesh=mesh,
        out_type=jax.ShapeDtypeStruct((BT // 2, 2 * D), jnp.float32),
        scratch_types=[
            pltpu.VMEM((M * CK,), jnp.int32),
            pltpu.VMEM((CK, D), jnp.float32),
            pltpu.VMEM((CK, D), jnp.float32),
            pltpu.VMEM((CK, D), jnp.float32),
            pltpu.VMEM((CK, D), jnp.float32),
            pltpu.SemaphoreType.DMA,
            pltpu.SemaphoreType.DMA,
            pltpu.SemaphoreType.DMA,
            pltpu.SemaphoreType.DMA,
            pltpu.SemaphoreType.DMA,
            pltpu.SemaphoreType.DMA,
            pltpu.SemaphoreType.DMA,
            pltpu.SemaphoreType.DMA,
        ],
        compiler_params=pltpu.CompilerParams(use_tc_tiling_on_sc=False),
    )
    def k(x_hbm, tok_hbm, out_hbm,
          idx_all, b0, b1, b2, b3,
          sg0, sg1, sg2, sg3, ss0, ss1, ss2, ss3):
        buf = (b0, b1, b2, b3)
        sg = (sg0, sg1, sg2, sg3)
        ss = (ss0, ss1, ss2, ss3)

        wid = lax.axis_index("s") * NC + lax.axis_index("c")

        def gather(c, b):
            return pltpu.make_async_copy(
                tok_hbm.at[idx_all.at[pl.ds(c * CK, CK)]], buf[b], sg[b])

        def store(c, s, i, b):
            row0 = CK * ((M // 2) * wid + 2 * i + (s // 2))
            return pltpu.make_async_copy(
                buf[b],
                out_hbm.at[pl.ds(row0, CK), pl.ds(D * (s % 2), D)],
                ss[b])

        pltpu.sync_copy(x_hbm.at[pl.ds(wid * (M * CK), M * CK)], idx_all)
        gather(0, 0).start()
        gather(1, 1).start()

        def body(i, carry):
            for s in range(NB):
                c = NB * i + s
                b = s
                gather(c, b).wait()

                @pl.when(c >= 2)
                def _():
                    store(c - 2, (s - 2) % NB, i - (1 if s < 2 else 0),
                          (s - 2) % NB).wait()

                @pl.when(c + 2 < M)
                def _():
                    gather(c + 2, (s + 2) % NB).start()

                store(c, s, i, b).start()
            return carry

        lax.fori_loop(0, M // NB, body, 0)
        store(M - 2, 2, M // NB - 1, 2).wait()
        store(M - 1, 3, M // NB - 1, 3).wait()

    return k


def _make_finish(B, T, D):
    TB = 8
    RB = TB * B // 2
    NP = RB // CK

    def body(g_ref, pos_ref, o_ref):
        v = g_ref[...]
        l3 = v[:, :D].reshape(NP, CK, D)
        r3 = v[:, D:].reshape(NP, CK, D)
        y = jnp.concatenate([l3, r3], axis=1)
        y = y.reshape(TB, B, D)
        z = jnp.swapaxes(y, 1, 2)
        o_ref[...] = z + pos_ref[...][:, :, None]

    return pl.pallas_call(
        body,
        grid=(T // TB,),
        in_specs=[
            pl.BlockSpec((RB, 2 * D), lambda i: (i, 0)),
            pl.BlockSpec((TB, D), lambda i: (i, 0)),
        ],
        out_specs=pl.BlockSpec((TB, D, B), lambda i: (i, 0, 0)),
        out_shape=jax.ShapeDtypeStruct((T, D, B), jnp.float32),
    )


def kernel(x, token_emb, pos_emb):
    B, T = x.shape
    V, D = token_emb.shape
    xt_flat = jnp.transpose(x).reshape(T * B).astype(jnp.int32)
    g = _make_gather(B, T, D, V)(xt_flat, token_emb)
    o = _make_finish(B, T, D)(g, pos_emb)
    return jnp.transpose(o, (2, 0, 1))


# speedup vs baseline: 7.6324x; 1.0581x over previous
import functools

import jax
import jax.numpy as jnp
from jax import lax
from jax.experimental import pallas as pl
from jax.experimental.pallas import tpu as pltpu
from jax.experimental.pallas import tpu_sc as plsc

CK = 128
NB = 4
NSPLIT = 2


def _make_gather(B, T, D, V, split):
    NC, NS = 2, 16
    NW = NC * NS
    BT = B * T
    ROWS = BT // NSPLIT
    assert ROWS % (NW * 2 * CK) == 0 and D * 2 == 128
    M = ROWS // (NW * CK)
    assert M % NB == 0 and M >= 2 * NB

    mesh = plsc.VectorSubcoreMesh(core_axis_name="c", subcore_axis_name="s")

    @functools.partial(
        pl.kernel,
        mesh=mesh,
        out_type=jax.ShapeDtypeStruct((ROWS // 2, 2 * D), jnp.float32),
        scratch_types=[
            pltpu.VMEM((M * CK,), jnp.int32),
            pltpu.VMEM((CK, D), jnp.float32),
            pltpu.VMEM((CK, D), jnp.float32),
            pltpu.VMEM((CK, D), jnp.float32),
            pltpu.VMEM((CK, D), jnp.float32),
            pltpu.SemaphoreType.DMA,
            pltpu.SemaphoreType.DMA,
            pltpu.SemaphoreType.DMA,
            pltpu.SemaphoreType.DMA,
            pltpu.SemaphoreType.DMA,
            pltpu.SemaphoreType.DMA,
            pltpu.SemaphoreType.DMA,
            pltpu.SemaphoreType.DMA,
        ],
        compiler_params=pltpu.CompilerParams(use_tc_tiling_on_sc=False),
    )
    def k(x_hbm, tok_hbm, out_hbm,
          idx_all, b0, b1, b2, b3,
          sg0, sg1, sg2, sg3, ss0, ss1, ss2, ss3):
        buf = (b0, b1, b2, b3)
        sg = (sg0, sg1, sg2, sg3)
        ss = (ss0, ss1, ss2, ss3)

        wid = lax.axis_index("s") * NC + lax.axis_index("c")

        def gather(c, b):
            return pltpu.make_async_copy(
                tok_hbm.at[idx_all.at[pl.ds(c * CK, CK)]], buf[b], sg[b])

        def store(c, s, i, b):
            row0 = CK * ((M // 2) * wid + 2 * i + (s // 2))
            return pltpu.make_async_copy(
                buf[b],
                out_hbm.at[pl.ds(row0, CK), pl.ds(D * (s % 2), D)],
                ss[b])

        pltpu.sync_copy(
            x_hbm.at[pl.ds(split * ROWS + wid * (M * CK), M * CK)], idx_all)
        gather(0, 0).start()
        gather(1, 1).start()

        def body(i, carry):
            for s in range(NB):
                c = NB * i + s
                b = s
                gather(c, b).wait()

                @pl.when(c >= 2)
                def _():
                    store(c - 2, (s - 2) % NB, i - (1 if s < 2 else 0),
                          (s - 2) % NB).wait()

                @pl.when(c + 2 < M)
                def _():
                    gather(c + 2, (s + 2) % NB).start()

                store(c, s, i, b).start()
            return carry

        lax.fori_loop(0, M // NB, body, 0)
        store(M - 2, 2, M // NB - 1, 2).wait()
        store(M - 1, 3, M // NB - 1, 3).wait()

    return k


def _make_finish(B, T, D, split, aliased):
    TS = T // NSPLIT
    TB = 4
    RB = TB * B // 2
    NP = RB // CK
    T0 = split * TS

    if aliased:
        def body(o_in_ref, g_ref, pos_ref, o_ref):
            del o_in_ref
            _finish_block(g_ref, pos_ref, o_ref, TB, B, D, NP)
        in_specs = [
            pl.BlockSpec(memory_space=pl.ANY),
            pl.BlockSpec((RB, 2 * D), lambda i: (i, 0)),
            pl.BlockSpec((1, TB, D), lambda i: (T0 // TB + i, 0, 0)),
        ]
        io_aliases = {0: 0}
    else:
        def body(g_ref, pos_ref, o_ref):
            _finish_block(g_ref, pos_ref, o_ref, TB, B, D, NP)
        in_specs = [
            pl.BlockSpec((RB, 2 * D), lambda i: (i, 0)),
            pl.BlockSpec((1, TB, D), lambda i: (T0 // TB + i, 0, 0)),
        ]
        io_aliases = {}

    return pl.pallas_call(
        body,
        grid=(TS // TB,),
        in_specs=in_specs,
        out_specs=pl.BlockSpec((TB, D, B), lambda i: (T0 // TB + i, 0, 0)),
        out_shape=jax.ShapeDtypeStruct((T, D, B), jnp.float32),
        input_output_aliases=io_aliases,
    )


def _finish_block(g_ref, pos_ref, o_ref, TB, B, D, NP):
    v = g_ref[...]
    l3 = v[:, :D].reshape(NP, CK, D)
    r3 = v[:, D:].reshape(NP, CK, D)
    y = jnp.concatenate([l3, r3], axis=1)
    y = y.reshape(TB, B, D)
    z = jnp.swapaxes(y, 1, 2)
    o_ref[...] = z + pos_ref[0][:, :, None]


def kernel(x, token_emb, pos_emb):
    B, T = x.shape
    V, D = token_emb.shape
    xt_flat = jnp.transpose(x).reshape(T * B).astype(jnp.int32)
    pos3 = pos_emb.reshape(T // 4, 4, D)
    g0 = _make_gather(B, T, D, V, 0)(xt_flat, token_emb)
    g1 = _make_gather(B, T, D, V, 1)(xt_flat, token_emb)
    o = _make_finish(B, T, D, 0, aliased=False)(g0, pos3)
    o = _make_finish(B, T, D, 1, aliased=True)(o, g1, pos3)
    return jnp.transpose(o, (2, 0, 1))


# speedup vs baseline: 7.6989x; 1.0087x over previous
import functools

import jax
import jax.numpy as jnp
from jax import lax
from jax.experimental import pallas as pl
from jax.experimental.pallas import tpu as pltpu
from jax.experimental.pallas import tpu_sc as plsc

CK = 128
NB = 5
NSPLIT = 2


def _prep_table(token_emb, V, D):
    tok_lin = jax.lax.optimization_barrier(jnp.reshape(token_emb, (V * D,)))
    return tok_lin.reshape(V, D)


def _make_gather(B, T, D, V, split):
    NC, NS = 2, 16
    NW = NC * NS
    BT = B * T
    ROWS = BT // NSPLIT
    assert ROWS % (NW * 2 * CK) == 0 and D * 2 == 128
    M = ROWS // (NW * CK)
    assert M % NB == 0 and M >= 2 * NB

    mesh = plsc.VectorSubcoreMesh(core_axis_name="c", subcore_axis_name="s")

    @functools.partial(
        pl.kernel,
        mesh=mesh,
        out_type=jax.ShapeDtypeStruct((ROWS // 2, 2 * D), jnp.float32),
        scratch_types=[
            pltpu.VMEM((M * CK,), jnp.int32),
            pltpu.VMEM((CK, D), jnp.float32),
            pltpu.VMEM((CK, D), jnp.float32),
            pltpu.VMEM((CK, D), jnp.float32),
            pltpu.VMEM((CK, D), jnp.float32),
            pltpu.VMEM((CK, D), jnp.float32),
            pltpu.SemaphoreType.DMA,
            pltpu.SemaphoreType.DMA,
            pltpu.SemaphoreType.DMA,
            pltpu.SemaphoreType.DMA,
            pltpu.SemaphoreType.DMA,
            pltpu.SemaphoreType.DMA,
            pltpu.SemaphoreType.DMA,
            pltpu.SemaphoreType.DMA,
            pltpu.SemaphoreType.DMA,
            pltpu.SemaphoreType.DMA,
        ],
        compiler_params=pltpu.CompilerParams(use_tc_tiling_on_sc=False),
    )
    def k(x_hbm, tok_hbm, out_hbm,
          idx_all, b0, b1, b2, b3, b4,
          sg0, sg1, sg2, sg3, sg4, ss0, ss1, ss2, ss3, ss4):
        buf = (b0, b1, b2, b3, b4)
        sg = (sg0, sg1, sg2, sg3, sg4)
        ss = (ss0, ss1, ss2, ss3, ss4)

        wid = lax.axis_index("s") * NC + lax.axis_index("c")

        def gather(c, b):
            return pltpu.make_async_copy(
                tok_hbm.at[idx_all.at[pl.ds(c * CK, CK)]], buf[b], sg[b])

        def store(c, b):
            row0 = CK * ((M // 2) * wid) + CK * (c // 2)
            col = D * (c % 2)
            return pltpu.make_async_copy(
                buf[b],
                out_hbm.at[pl.ds(row0, CK), pl.ds(col, D)],
                ss[b])

        pltpu.sync_copy(
            x_hbm.at[pl.ds(split * ROWS + wid * (M * CK), M * CK)], idx_all)
        gather(0, 0).start()
        gather(1, 1).start()
        gather(2, 2).start()

        def body(i, carry):
            for s in range(NB):
                c = NB * i + s
                b = s
                gather(c, b).wait()

                @pl.when(c >= 2)
                def _():
                    store(c - 2, (s - 2) % NB).wait()

                @pl.when(c + 3 < M)
                def _():
                    gather(c + 3, (s + 3) % NB).start()

                store(c, b).start()
            return carry

        lax.fori_loop(0, M // NB, body, 0)
        store(M - 2, (M - 2) % NB).wait()
        store(M - 1, (M - 1) % NB).wait()

    return k


def _make_finish(B, T, D, split, aliased):
    TS = T // NSPLIT
    TB = 4
    RB = TB * B // 2
    NP = RB // CK
    T0 = split * TS

    if aliased:
        def body(o_in_ref, g_ref, pos_ref, o_ref):
            del o_in_ref
            _finish_block(g_ref, pos_ref, o_ref, TB, B, D, NP)
        in_specs = [
            pl.BlockSpec(memory_space=pl.ANY),
            pl.BlockSpec((RB, 2 * D), lambda i: (i, 0)),
            pl.BlockSpec((1, TB, D), lambda i: (T0 // TB + i, 0, 0)),
        ]
        io_aliases = {0: 0}
    else:
        def body(g_ref, pos_ref, o_ref):
            _finish_block(g_ref, pos_ref, o_ref, TB, B, D, NP)
        in_specs = [
            pl.BlockSpec((RB, 2 * D), lambda i: (i, 0)),
            pl.BlockSpec((1, TB, D), lambda i: (T0 // TB + i, 0, 0)),
        ]
        io_aliases = {}

    return pl.pallas_call(
        body,
        grid=(TS // TB,),
        in_specs=in_specs,
        out_specs=pl.BlockSpec((TB, D, B), lambda i: (T0 // TB + i, 0, 0)),
        out_shape=jax.ShapeDtypeStruct((T, D, B), jnp.float32),
        input_output_aliases=io_aliases,
    )


def _finish_block(g_ref, pos_ref, o_ref, TB, B, D, NP):
    v = g_ref[...]
    l3 = v[:, :D].reshape(NP, CK, D)
    r3 = v[:, D:].reshape(NP, CK, D)
    y = jnp.concatenate([l3, r3], axis=1)
    y = y.reshape(TB, B, D)
    z = jnp.swapaxes(y, 1, 2)
    o_ref[...] = z + pos_ref[0][:, :, None]


def kernel(x, token_emb, pos_emb):
    B, T = x.shape
    V, D = token_emb.shape
    xt_flat = jnp.transpose(x).reshape(T * B).astype(jnp.int32)
    pos3 = pos_emb.reshape(T // 4, 4, D)
    tok_rows = _prep_table(token_emb, V, D)
    g0 = _make_gather(B, T, D, V, 0)(xt_flat, tok_rows)
    g1 = _make_gather(B, T, D, V, 1)(xt_flat, tok_rows)
    o = _make_finish(B, T, D, 0, aliased=False)(g0, pos3)
    o = _make_finish(B, T, D, 1, aliased=True)(o, g1, pos3)
    return jnp.transpose(o, (2, 0, 1))
